# baseline (speedup 1.0000x reference)
import functools

import jax
import jax.numpy as jnp
from jax import lax
from jax.experimental import pallas as pl
from jax.experimental.pallas import tpu as pltpu
from jax.experimental.pallas import tpu_sc as plsc

_INFO = plsc.get_sparse_core_info()
_NC, _NS, _L = _INFO.num_cores, _INFO.num_subcores, _INFO.num_lanes
_NW = _NC * _NS
_CH = 128


def _make_sc_partials(B, V, S):
    rpt = B // _NW
    nch = S // _CH
    mesh = plsc.VectorSubcoreMesh(core_axis_name="c", subcore_axis_name="s")

    @functools.partial(
        pl.kernel,
        out_type=(
            jax.ShapeDtypeStruct((B,), jnp.float32),
            jax.ShapeDtypeStruct((B,), jnp.float32),
        ),
        mesh=mesh,
        compiler_params=pltpu.CompilerParams(needs_layout_passes=False),
        scratch_types=[
            pltpu.VMEM((rpt,), jnp.int32),
            pltpu.VMEM((rpt,), jnp.float32),
            pltpu.VMEM((nch, _CH), jnp.int32),
            pltpu.VMEM((nch, _CH), jnp.int32),
            pltpu.VMEM((nch, _CH), jnp.float32),
            pltpu.VMEM((rpt,), jnp.float32),
            pltpu.VMEM((rpt,), jnp.float32),
            pltpu.SemaphoreType.DMA,
        ],
    )
    def sc_partials(flat_hbm, tgt_hbm, smp_hbm, outA_hbm, outP_hbm,
                    tgi_v, tsc_v, sidx_v, ridx_v, s_v, oA_v, oP_v, sem):
        wid = lax.axis_index("s") * _NC + lax.axis_index("c")
        base = wid * rpt
        lane0 = lax.iota(jnp.int32, _L) == 0

        pltpu.sync_copy(tgt_hbm.at[pl.ds(base, rpt)], tgi_v)
        for k in range(rpt // _L):
            tg = tgi_v[pl.ds(k * _L, _L)]
            rows = base + k * _L + lax.iota(jnp.int32, _L)
            flat = tg + rows * V
            pltpu.async_copy(flat_hbm.at[flat],
                             tsc_v.at[pl.ds(k * _L, _L)], sem).wait()

        def row_step(r, carry):
            b = base + r
            pltpu.sync_copy(smp_hbm.at[b], sidx_v)
            bv = jnp.full((_L,), b * V, jnp.int32)

            def fidx(c, _):
                for i in range(_CH // _L):
                    sl = pl.ds(i * _L, _L)
                    ridx_v[c, sl] = sidx_v[c, sl] + bv
                return 0
            lax.fori_loop(0, nch, fidx, 0)

            for c in range(nch):
                pltpu.async_copy(flat_hbm.at[ridx_v.at[c]], s_v.at[c], sem)
            for c in range(nch):
                pltpu.make_async_copy(flat_hbm.at[ridx_v.at[c]],
                                      s_v.at[c], sem).wait()

            rvec = jnp.full((_L,), r, jnp.int32)
            tvec = plsc.load_gather(tsc_v, [rvec])

            def p1(c, mvec):
                for i in range(_CH // _L):
                    mvec = jnp.maximum(mvec, s_v[c, pl.ds(i * _L, _L)])
                return mvec
            mvec = lax.fori_loop(0, nch, p1,
                                 jnp.full((_L,), -jnp.inf, jnp.float32))
            m = lax.reduce_max(mvec, (0,))

            zero = jnp.zeros((_L,), jnp.float32)

            def p2(c, acc):
                accE, accA, accP = acc
                for i in range(_CH // _L):
                    v = s_v[c, pl.ds(i * _L, _L)]
                    e = jnp.exp(v - m)
                    sig = 1.0 / (1.0 + jnp.exp(v - tvec))
                    accE = accE + e
                    accA = accA + e * sig
                    accP = accP + e * v * v
                return (accE, accA, accP)
            accE, accA, accP = lax.fori_loop(0, nch, p2, (zero, zero, zero))

            E = lax.reduce_sum(accE, (0,))
            A = lax.reduce_sum(accA, (0,))
            P = lax.reduce_sum(accP, (0,))
            Evec = jnp.full((_L,), E)
            plsc.store_scatter(oA_v, [rvec], jnp.full((_L,), A) / Evec,
                               mask=lane0)
            plsc.store_scatter(oP_v, [rvec], jnp.full((_L,), P) / Evec,
                               mask=lane0)
            return carry

        lax.fori_loop(0, rpt, row_step, 0)
        pltpu.sync_copy(oA_v, outA_hbm.at[pl.ds(base, rpt)])
        pltpu.sync_copy(oP_v, outP_hbm.at[pl.ds(base, rpt)])

    return sc_partials


def _finish(a, p):
    B = a.shape[0]
    a2 = a.reshape(8, B // 8)
    p2 = p.reshape(8, B // 8)

    def body(a_ref, p_ref, o_ref):
        o_ref[0, 0] = jnp.mean(-jnp.log(a_ref[...]) + p_ref[...])

    out = pl.pallas_call(
        body,
        out_shape=jax.ShapeDtypeStruct((1, 1), jnp.float32),
        out_specs=pl.BlockSpec(memory_space=pltpu.SMEM),
    )(a2, p2)
    return out[0, 0]


def kernel(input, target, samples):
    B, V = input.shape
    S = samples.shape[1]
    tgt = target.astype(jnp.int32)
    smp = samples.astype(jnp.int32).reshape(B, S // _CH, _CH)
    flat = input.reshape(B * V)
    outA, outP = _make_sc_partials(B, V, S)(flat, tgt, smp)
    return _finish(outA, outP)
